# baseline (speedup 1.0000x reference)
import functools
import math

import jax
import jax.numpy as jnp
import numpy as np
from jax import lax
from jax.experimental import pallas as pl
from jax.experimental.pallas import tpu as pltpu
from jax.experimental.pallas import tpu_sc as plsc

_VOCAB = 1000000
_D = 64
_BATCH = 16384
_SEQ = 50

_NW = 32
_ROWS = _BATCH * _SEQ
_PER_W = _ROWS // _NW
_XROW = _PER_W // 128
_CSEQ = 4
_C = _CSEQ * _SEQ
_NCHUNK = _PER_W // _C
_SUBS = (80, 80, 40)
_SCALE = 8.0
_BB = 128


def _pos_encoding():
    positions = np.arange(_SEQ)[:, np.newaxis]
    div_term = np.exp(np.arange(0, _D, 2) * -(np.log(10000.0) / _D))
    angle_rads = positions * div_term
    pe = np.zeros((_SEQ, _D), dtype=np.float32)
    pe[:, 0::2] = np.sin(angle_rads)
    pe[:, 1::2] = np.cos(angle_rads)
    return pe


_PE = _pos_encoding()


def _make_sc_kernel():
    mesh = plsc.VectorSubcoreMesh(core_axis_name="c", subcore_axis_name="s")

    @functools.partial(
        pl.kernel,
        out_type=jax.ShapeDtypeStruct((_ROWS // 2, 128), jnp.float32),
        mesh=mesh,
        compiler_params=pltpu.CompilerParams(use_tc_tiling_on_sc=False),
        scratch_types=[
            pltpu.VMEM((_XROW, 128), jnp.int32),
            pltpu.VMEM((_PER_W,), jnp.int32),
            pltpu.VMEM((2, _C, _D), jnp.float32),
            pltpu.VMEM((_C // 2, 128), jnp.float32),
            pltpu.VMEM((_SEQ, _D), jnp.float32),
            pltpu.SemaphoreType.DMA,
            pltpu.SemaphoreType.DMA,
        ],
    )
    def sc_kernel(x_hbm, pe_hbm, table_hbm, out_hbm, xs_v, idx_v, rows_v, ob_v,
                  pe_v, g0, g1):
        wid = lax.axis_index("s") * 2 + lax.axis_index("c")

        pltpu.sync_copy(x_hbm.at[pl.ds(wid * _XROW, _XROW)], xs_v)
        pltpu.sync_copy(pe_hbm, pe_v)

        def repack_body(r, carry):
            for j in range(128 // 16):
                idx_v[pl.ds(r * 128 + 16 * j, 16)] = xs_v[r, pl.ds(16 * j, 16)]
            return carry

        lax.fori_loop(0, _XROW, repack_body, 0)

        sems = (g0, g1)

        def sub_copies(c, b):
            copies = []
            off = 0
            for sub in _SUBS:
                copies.append((
                    table_hbm.at[idx_v.at[pl.ds(c * _C + off, sub)]],
                    rows_v.at[b, pl.ds(off, sub)],
                ))
                off += sub
            return copies

        def fire(c, b):
            for src, dst in sub_copies(c, b):
                pltpu.async_copy(src, dst, sems[b])

        def finish(c, b):
            for src, dst in sub_copies(c, b):
                pltpu.make_async_copy(src, dst, sems[b]).wait()

            def seq_body(q, carry):
                def row_body(r, carry2):
                    i = q * _SEQ + r
                    half = i // 2
                    col0 = (i % 2) * _D
                    for j in range(_D // 16):
                        ob_v[half, pl.ds(col0 + 16 * j, 16)] = (
                            rows_v[b, i, pl.ds(16 * j, 16)] * _SCALE
                            + pe_v[r, pl.ds(16 * j, 16)]
                        )
                    return carry2

                lax.fori_loop(0, _SEQ, row_body, 0)
                return carry

            lax.fori_loop(0, _CSEQ, seq_body, 0)

            h0 = (wid * _NCHUNK + c) * (_C // 2)
            pltpu.sync_copy(ob_v, out_hbm.at[pl.ds(h0, _C // 2)])

        fire(0, 0)

        def loop_body(c2, carry):
            c0 = 2 * c2

            fire(c0 + 1, 1)
            finish(c0, 0)

            @pl.when(c0 + 2 < _NCHUNK)
            def _():
                fire(c0 + 2, 0)

            finish(c0 + 1, 1)
            return carry

        lax.fori_loop(0, _NCHUNK // 2, loop_body, 0)

    return sc_kernel


_sc_kernel = _make_sc_kernel()


def _relayout_body(in_ref, out_ref):
    a = in_ref[...]
    pair = jnp.stack([a[:, :_D], a[:, _D:]], axis=1)
    out_ref[...] = pair.reshape(_BB, _SEQ, _D)


_relayout = pl.pallas_call(
    _relayout_body,
    grid=(_BATCH // _BB,),
    in_specs=[
        pl.BlockSpec((_BB * _SEQ * _D // 128, 128), lambda i: (i, 0)),
    ],
    out_specs=pl.BlockSpec((_BB, _SEQ, _D), lambda i: (i, 0, 0)),
    out_shape=jax.ShapeDtypeStruct((_BATCH, _SEQ, _D), jnp.float32),
)


@jax.jit
def kernel(x, table):
    x128 = x.reshape(_ROWS // 128, 128)
    pe = jnp.asarray(_PE)
    packed = _sc_kernel(x128, pe, table)
    return _relayout(packed)


# speedup vs baseline: 1.2186x; 1.2186x over previous
import functools
import math

import jax
import jax.numpy as jnp
import numpy as np
from jax import lax
from jax.experimental import pallas as pl
from jax.experimental.pallas import tpu as pltpu
from jax.experimental.pallas import tpu_sc as plsc

_VOCAB = 1000000
_D = 64
_BATCH = 16384
_SEQ = 50

_NW = 32
_ROWS = _BATCH * _SEQ
_PER_W = _ROWS // _NW
_XROW = _PER_W // 128
_CSEQ = 4
_C = _CSEQ * _SEQ
_NCHUNK = _PER_W // _C
_SUBS = (80, 80, 40)
_SCALE = 8.0
_TNB = 2048


def _pos_encoding():
    positions = np.arange(_SEQ)[:, np.newaxis]
    div_term = np.exp(np.arange(0, _D, 2) * -(np.log(10000.0) / _D))
    angle_rads = positions * div_term
    pe = np.zeros((_SEQ, _D), dtype=np.float32)
    pe[:, 0::2] = np.sin(angle_rads)
    pe[:, 1::2] = np.cos(angle_rads)
    return pe


_PE = _pos_encoding()


def _make_sc_kernel():
    mesh = plsc.VectorSubcoreMesh(core_axis_name="c", subcore_axis_name="s")

    @functools.partial(
        pl.kernel,
        out_type=jax.ShapeDtypeStruct((_BATCH, _SEQ, _D), jnp.float32),
        mesh=mesh,
        compiler_params=pltpu.CompilerParams(use_tc_tiling_on_sc=False),
        scratch_types=[
            pltpu.VMEM((_XROW, 128), jnp.int32),
            pltpu.VMEM((_PER_W,), jnp.int32),
            pltpu.VMEM((2, _C, 128), jnp.float32),
            pltpu.VMEM((_CSEQ, _SEQ, _D), jnp.float32),
            pltpu.VMEM((_SEQ, _D), jnp.float32),
            pltpu.SemaphoreType.DMA,
            pltpu.SemaphoreType.DMA,
        ],
    )
    def sc_kernel(x_hbm, pe_hbm, table_hbm, out_hbm, xs_v, idx_v, rows_v, ob_v,
                  pe_v, g0, g1):
        wid = lax.axis_index("s") * 2 + lax.axis_index("c")

        pltpu.sync_copy(x_hbm.at[pl.ds(wid * _XROW, _XROW)], xs_v)
        pltpu.sync_copy(pe_hbm, pe_v)

        def repack_body(r, carry):
            for j in range(128 // 16):
                idx_v[pl.ds(r * 128 + 16 * j, 16)] = xs_v[r, pl.ds(16 * j, 16)]
            return carry

        lax.fori_loop(0, _XROW, repack_body, 0)

        sems = (g0, g1)

        def sub_copies(c, b):
            copies = []
            off = 0
            for sub in _SUBS:
                copies.append((
                    table_hbm.at[idx_v.at[pl.ds(c * _C + off, sub)]],
                    rows_v.at[b, pl.ds(off, sub)],
                ))
                off += sub
            return copies

        def fire(c, b):
            for src, dst in sub_copies(c, b):
                pltpu.async_copy(src, dst, sems[b])

        def finish(c, b):
            for src, dst in sub_copies(c, b):
                pltpu.make_async_copy(src, dst, sems[b]).wait()

            def seq_body(q, carry):
                def row_body(r, carry2):
                    i = q * _SEQ + r
                    for j in range(_D // 16):
                        sl = pl.ds(16 * j, 16)
                        ob_v[q, r, sl] = rows_v[b, i, sl] * _SCALE + pe_v[r, sl]
                    return carry2

                lax.fori_loop(0, _SEQ, row_body, 0)
                return carry

            lax.fori_loop(0, _CSEQ, seq_body, 0)

            b0 = (wid * _NCHUNK + c) * _CSEQ
            pltpu.sync_copy(ob_v, out_hbm.at[pl.ds(b0, _CSEQ)])

        fire(0, 0)

        def loop_body(c2, carry):
            c0 = 2 * c2

            fire(c0 + 1, 1)
            finish(c0, 0)

            @pl.when(c0 + 2 < _NCHUNK)
            def _():
                fire(c0 + 2, 0)

            finish(c0 + 1, 1)
            return carry

        lax.fori_loop(0, _NCHUNK // 2, loop_body, 0)

    return sc_kernel


_sc_kernel = _make_sc_kernel()


def _transpose_body(in_ref, out_ref):
    out_ref[:, :_D] = in_ref[...].T


_transpose_table = pl.pallas_call(
    _transpose_body,
    grid=((_VOCAB + _TNB - 1) // _TNB,),
    in_specs=[pl.BlockSpec((_D, _TNB), lambda i: (0, i))],
    out_specs=pl.BlockSpec((_TNB, 128), lambda i: (i, 0)),
    out_shape=jax.ShapeDtypeStruct((_VOCAB, 128), jnp.float32),
)


@jax.jit
def kernel(x, table):
    x128 = x.reshape(_ROWS // 128, 128)
    pe = jnp.asarray(_PE)
    table128 = _transpose_table(table.T)
    return _sc_kernel(x128, pe, table128)


# speedup vs baseline: 1.5945x; 1.3085x over previous
import functools
import math

import jax
import jax.numpy as jnp
import numpy as np
from jax import lax
from jax.experimental import pallas as pl
from jax.experimental.pallas import tpu as pltpu
from jax.experimental.pallas import tpu_sc as plsc

_VOCAB = 1000000
_D = 64
_BATCH = 16384
_SEQ = 50

_NW = 32
_ROWS = _BATCH * _SEQ
_PER_W = _ROWS // _NW
_XROW = _PER_W // 128
_CSEQ = 4
_C = _CSEQ * _SEQ
_NCHUNK = _PER_W // _C
_SUBS = (80, 80, 40)
_SCALE = 8.0


def _pos_encoding():
    positions = np.arange(_SEQ)[:, np.newaxis]
    div_term = np.exp(np.arange(0, _D, 2) * -(np.log(10000.0) / _D))
    angle_rads = positions * div_term
    pe = np.zeros((_SEQ, _D), dtype=np.float32)
    pe[:, 0::2] = np.sin(angle_rads)
    pe[:, 1::2] = np.cos(angle_rads)
    return pe


_PE = _pos_encoding()


def _make_sc_kernel():
    mesh = plsc.VectorSubcoreMesh(core_axis_name="c", subcore_axis_name="s")

    @functools.partial(
        pl.kernel,
        out_type=jax.ShapeDtypeStruct((_BATCH, _SEQ, _D), jnp.float32),
        mesh=mesh,
        compiler_params=pltpu.CompilerParams(use_tc_tiling_on_sc=False),
        scratch_types=[
            pltpu.VMEM((_XROW, 128), jnp.int32),
            pltpu.VMEM((_PER_W,), jnp.int32),
            pltpu.VMEM((2, _C, _D), jnp.float32),
            pltpu.VMEM((2, _CSEQ, _SEQ, _D), jnp.float32),
            pltpu.VMEM((_SEQ, _D), jnp.float32),
            pltpu.SemaphoreType.DMA,
            pltpu.SemaphoreType.DMA,
            pltpu.SemaphoreType.DMA,
            pltpu.SemaphoreType.DMA,
        ],
    )
    def sc_kernel(x_hbm, pe_hbm, table_hbm, out_hbm, xs_v, idx_v, rows_v, ob_v,
                  pe_v, g0, g1, w0, w1):
        wid = lax.axis_index("s") * 2 + lax.axis_index("c")

        pltpu.sync_copy(x_hbm.at[pl.ds(wid * _XROW, _XROW)], xs_v)
        pltpu.sync_copy(pe_hbm, pe_v)

        def repack_body(r, carry):
            for j in range(128 // 16):
                idx_v[pl.ds(r * 128 + 16 * j, 16)] = xs_v[r, pl.ds(16 * j, 16)]
            return carry

        lax.fori_loop(0, _XROW, repack_body, 0)

        sems = (g0, g1)
        wsems = (w0, w1)

        def out_slice(c):
            b0 = (wid * _NCHUNK + c) * _CSEQ
            return out_hbm.at[pl.ds(b0, _CSEQ)]

        def sub_copies(c, b):
            copies = []
            off = 0
            for sub in _SUBS:
                copies.append((
                    table_hbm.at[idx_v.at[pl.ds(c * _C + off, sub)]],
                    rows_v.at[b, pl.ds(off, sub)],
                ))
                off += sub
            return copies

        def fire(c, b):
            for src, dst in sub_copies(c, b):
                pltpu.async_copy(src, dst, sems[b])

        def finish(c, b):
            for src, dst in sub_copies(c, b):
                pltpu.make_async_copy(src, dst, sems[b]).wait()

            @pl.when(c >= 2)
            def _():
                pltpu.make_async_copy(ob_v.at[b], out_slice(c - 2), wsems[b]).wait()

            def seq_body(q, carry):
                def row_body(r, carry2):
                    i = q * _SEQ + r
                    for j in range(_D // 16):
                        sl = pl.ds(16 * j, 16)
                        ob_v[b, q, r, sl] = rows_v[b, i, sl] * _SCALE + pe_v[r, sl]
                    return carry2

                lax.fori_loop(0, _SEQ, row_body, 0)
                return carry

            lax.fori_loop(0, _CSEQ, seq_body, 0)

            pltpu.async_copy(ob_v.at[b], out_slice(c), wsems[b])

        fire(0, 0)

        def loop_body(c2, carry):
            c0 = 2 * c2

            fire(c0 + 1, 1)
            finish(c0, 0)

            @pl.when(c0 + 2 < _NCHUNK)
            def _():
                fire(c0 + 2, 0)

            finish(c0 + 1, 1)
            return carry

        lax.fori_loop(0, _NCHUNK // 2, loop_body, 0)

        pltpu.make_async_copy(ob_v.at[0], out_slice(_NCHUNK - 2), wsems[0]).wait()
        pltpu.make_async_copy(ob_v.at[1], out_slice(_NCHUNK - 1), wsems[1]).wait()

    return sc_kernel


_sc_kernel = _make_sc_kernel()


@jax.jit
def kernel(x, table):
    x128 = x.reshape(_ROWS // 128, 128)
    pe = jnp.asarray(_PE)
    return _sc_kernel(x128, pe, table)
